# Initial kernel scaffold; baseline (speedup 1.0000x reference)
import functools

import jax
import jax.numpy as jnp
from jax import lax
from jax.experimental import pallas as pl
from jax.experimental.pallas import tpu as pltpu
from jax.experimental.pallas import tpu_sc as plsc

N = 10000
E = 320000
HID = 16
NPAD = 10016
NW = 32
W = 128
K = 79
EPAD = NW * K * W
ROWS_PER_SUB = NPAD // 16

_mesh = plsc.VectorSubcoreMesh(core_axis_name="c", subcore_axis_name="s")


@functools.partial(
    pl.kernel,
    out_type=jax.ShapeDtypeStruct((2, NPAD), jnp.float32),
    mesh=_mesh,
    scratch_types=[
        pltpu.VMEM((K, W), jnp.int32),
        pltpu.VMEM((W,), jnp.float32),
        pltpu.VMEM_SHARED((NPAD,), jnp.float32),
    ],
)
def _deg_kernel(dst_hbm, zeros_n_hbm, deg_out, dstidx_v, ones_v, deg_sh):
    c = lax.axis_index("c")
    s = lax.axis_index("s")
    wid = c * 16 + s
    base = s * ROWS_PER_SUB
    pltpu.sync_copy(zeros_n_hbm.at[pl.ds(base, ROWS_PER_SUB)],
                    deg_sh.at[pl.ds(base, ROWS_PER_SUB)])
    for o in range(W // 16):
        ones_v[pl.ds(o * 16, 16)] = jnp.ones((16,), jnp.float32)
    pltpu.sync_copy(dst_hbm.at[wid], dstidx_v)
    plsc.subcore_barrier()

    def body(j, carry):
        pltpu.sync_copy(ones_v, deg_sh.at[dstidx_v.at[j]], add=True)
        return carry

    lax.fori_loop(0, K, body, 0)
    plsc.subcore_barrier()
    pltpu.sync_copy(deg_sh.at[pl.ds(base, ROWS_PER_SUB)],
                    deg_out.at[c, pl.ds(base, ROWS_PER_SUB)])


def _tc1_body(deg_ref, x_ref, w1_ref, dis_ref, hp_ref):
    deg = deg_ref[0, :] + deg_ref[1, :] + 1.0
    dis = lax.rsqrt(deg)
    dis_ref[...] = dis
    h = jnp.dot(x_ref[...], w1_ref[...], preferred_element_type=jnp.float32)
    hp_ref[0:N, :] = h * dis[0:N, None]
    hp_ref[N:NPAD, :] = jnp.zeros((NPAD - N, HID), jnp.float32)


def _tc1(deg_part, x, w1):
    return pl.pallas_call(
        _tc1_body,
        out_shape=(
            jax.ShapeDtypeStruct((NPAD,), jnp.float32),
            jax.ShapeDtypeStruct((NPAD, HID), jnp.float32),
        ),
    )(deg_part, x, w1)


@functools.partial(
    pl.kernel,
    out_type=(
        jax.ShapeDtypeStruct((2, NPAD, HID), jnp.float32),
        jax.ShapeDtypeStruct((2, NPAD), jnp.float32),
    ),
    mesh=_mesh,
    scratch_types=[
        pltpu.VMEM((K, W), jnp.int32),
        pltpu.VMEM((K, W), jnp.int32),
        pltpu.VMEM((NPAD,), jnp.float32),
        pltpu.VMEM((K, W), jnp.float32),
        pltpu.VMEM((W, HID), jnp.float32),
        pltpu.VMEM_SHARED((NPAD, HID), jnp.float32),
        pltpu.VMEM_SHARED((NPAD,), jnp.float32),
        pltpu.SemaphoreType.DMA,
    ],
)
def _main_kernel(src_hbm, dst_hbm, hp_hbm, dis_hbm, zeros_nk_hbm, zeros_n_hbm,
                 acc_out, cpre_out,
                 srcidx_v, dstidx_v, dis_v, vals_v, rows_v, acc_sh, cpre_sh, sem):
    c = lax.axis_index("c")
    s = lax.axis_index("s")
    wid = c * 16 + s
    base = s * ROWS_PER_SUB
    pltpu.sync_copy(zeros_nk_hbm.at[pl.ds(base, ROWS_PER_SUB)],
                    acc_sh.at[pl.ds(base, ROWS_PER_SUB)])
    pltpu.sync_copy(zeros_n_hbm.at[pl.ds(base, ROWS_PER_SUB)],
                    cpre_sh.at[pl.ds(base, ROWS_PER_SUB)])
    pltpu.sync_copy(src_hbm.at[wid], srcidx_v)
    pltpu.sync_copy(dst_hbm.at[wid], dstidx_v)
    pltpu.sync_copy(dis_hbm, dis_v)

    def gat(j, carry):
        for o in range(W // 16):
            dv = dstidx_v[j, pl.ds(o * 16, 16)]
            vals_v[j, pl.ds(o * 16, 16)] = plsc.load_gather(dis_v, [dv])
        return carry

    lax.fori_loop(0, K, gat, 0)
    plsc.subcore_barrier()

    def body(j, carry):
        pltpu.async_copy(hp_hbm.at[srcidx_v.at[j]], rows_v, sem).wait()
        pltpu.sync_copy(rows_v, acc_sh.at[dstidx_v.at[j]], add=True)
        pltpu.sync_copy(vals_v.at[j], cpre_sh.at[srcidx_v.at[j]], add=True)
        return carry

    lax.fori_loop(0, K, body, 0)
    plsc.subcore_barrier()
    pltpu.sync_copy(acc_sh.at[pl.ds(base, ROWS_PER_SUB)],
                    acc_out.at[c, pl.ds(base, ROWS_PER_SUB)])
    pltpu.sync_copy(cpre_sh.at[pl.ds(base, ROWS_PER_SUB)],
                    cpre_out.at[c, pl.ds(base, ROWS_PER_SUB)])


def _tc2_body(acc_ref, cpre_ref, hp_ref, dis_ref, b1_ref, w2_ref, b2_ref,
              out_ref):
    dis = dis_ref[...]
    sacc = acc_ref[0] + acc_ref[1] + hp_ref[...]
    out1 = sacc[0:N, :] * dis[0:N, None] + b1_ref[...][None, :]
    r = jnp.maximum(out1, 0.0)
    cpre = cpre_ref[0] + cpre_ref[1] + dis
    cw = dis * cpre
    v = jnp.sum(r * cw[0:N, None], axis=0)
    out = jnp.sum(w2_ref[...] * v[:, None], axis=0) * (1.0 / N) + b2_ref[...]
    out_ref[...] = out


def _tc2(acc_part, cpre_part, hp, dis, b1, w2, b2):
    return pl.pallas_call(
        _tc2_body,
        out_shape=jax.ShapeDtypeStruct((HID,), jnp.float32),
    )(acc_part, cpre_part, hp, dis, b1, w2, b2)


def kernel(neigborhood_state, edges, W1, b1, W2, b2):
    src = edges[0].astype(jnp.int32)
    dst = edges[1].astype(jnp.int32)
    pad = N + (jnp.arange(EPAD - E, dtype=jnp.int32) % 16)
    src_p = jnp.concatenate([src, pad]).reshape(NW, K, W)
    dst_p = jnp.concatenate([dst, pad]).reshape(NW, K, W)
    zeros_n = jnp.zeros((NPAD,), jnp.float32)
    zeros_nk = jnp.zeros((NPAD, HID), jnp.float32)

    deg_part = _deg_kernel(dst_p, zeros_n)
    dis, hp = _tc1(deg_part, neigborhood_state, W1)
    acc_part, cpre_part = _main_kernel(src_p, dst_p, hp, dis, zeros_nk, zeros_n)
    return _tc2(acc_part, cpre_part, hp, dis, b1, W2, b2)


# speedup vs baseline: 60.8888x; 60.8888x over previous
import functools

import jax
import jax.numpy as jnp
from jax import lax
from jax.experimental import pallas as pl
from jax.experimental.pallas import tpu as pltpu
from jax.experimental.pallas import tpu_sc as plsc

N = 10000
E = 320000
HID = 16
NPAD = 10112
NW = 32
W = 128
K = 79
EPAD = NW * K * W
ROWS_PER_SUB = NPAD // 16

_mesh = plsc.VectorSubcoreMesh(core_axis_name="c", subcore_axis_name="s")


@functools.partial(
    pl.kernel,
    out_type=jax.ShapeDtypeStruct((2 * NPAD,), jnp.float32),
    mesh=_mesh,
    scratch_types=[
        pltpu.VMEM((K, W), jnp.int32),
        pltpu.VMEM((W,), jnp.float32),
        pltpu.VMEM((640,), jnp.float32),
        pltpu.VMEM_SHARED((NPAD,), jnp.float32),
    ],
)
def _deg_kernel(dst_hbm, deg_out, dstidx_v, ones_v, zbuf, deg_sh):
    c = lax.axis_index("c")
    s = lax.axis_index("s")
    wid = c * 16 + s
    base = s * ROWS_PER_SUB
    for o in range(W // 16):
        ones_v[pl.ds(o * 16, 16)] = jnp.ones((16,), jnp.float32)

    def zstore(i, carry):
        zbuf[pl.ds(i * 16, 16)] = jnp.zeros((16,), jnp.float32)
        return carry

    lax.fori_loop(0, 40, zstore, 0)
    pltpu.sync_copy(zbuf.at[pl.ds(0, ROWS_PER_SUB)],
                    deg_sh.at[pl.ds(base, ROWS_PER_SUB)])
    pltpu.sync_copy(dst_hbm.at[wid], dstidx_v)
    plsc.subcore_barrier()

    def body(j, carry):
        pltpu.sync_copy(ones_v, deg_sh.at[dstidx_v.at[j]], add=True)
        return carry

    lax.fori_loop(0, K, body, 0)
    plsc.subcore_barrier()
    pltpu.sync_copy(deg_sh.at[pl.ds(base, ROWS_PER_SUB)],
                    zbuf.at[pl.ds(0, ROWS_PER_SUB)])
    pltpu.sync_copy(zbuf.at[pl.ds(0, ROWS_PER_SUB)],
                    deg_out.at[pl.ds(c * NPAD + base, ROWS_PER_SUB)])


def _tc1_body(deg_ref, x_ref, w1_ref, dis_ref, hp_ref):
    deg = deg_ref[0, :] + deg_ref[1, :] + 1.0
    dis = lax.rsqrt(deg)
    dis_ref[...] = dis
    h = jnp.dot(x_ref[...], w1_ref[...], preferred_element_type=jnp.float32)
    hp_ref[0:N, :] = h * dis[0:N, None]
    hp_ref[N:NPAD, :] = jnp.zeros((NPAD - N, HID), jnp.float32)


def _tc1(deg_part, x, w1):
    return pl.pallas_call(
        _tc1_body,
        out_shape=(
            jax.ShapeDtypeStruct((NPAD,), jnp.float32),
            jax.ShapeDtypeStruct((NPAD, HID), jnp.float32),
        ),
    )(deg_part, x, w1)


@functools.partial(
    pl.kernel,
    out_type=(
        jax.ShapeDtypeStruct((2 * NPAD, HID), jnp.float32),
        jax.ShapeDtypeStruct((2 * NPAD,), jnp.float32),
    ),
    mesh=_mesh,
    scratch_types=[
        pltpu.VMEM((K, W), jnp.int32),
        pltpu.VMEM((K, W), jnp.int32),
        pltpu.VMEM((W,), jnp.float32),
        pltpu.VMEM((W, HID), jnp.float32),
        pltpu.VMEM((ROWS_PER_SUB, HID), jnp.float32),
        pltpu.VMEM((640,), jnp.float32),
        pltpu.VMEM_SHARED((NPAD, HID), jnp.float32),
        pltpu.VMEM_SHARED((NPAD,), jnp.float32),
        pltpu.VMEM_SHARED((NPAD,), jnp.float32),
        pltpu.SemaphoreType.DMA,
        pltpu.SemaphoreType.DMA,
    ],
    compiler_params=pltpu.CompilerParams(use_tc_tiling_on_sc=False),
)
def _main_kernel(src_hbm, dst_hbm, hp_hbm, dis_hbm,
                 acc_out, cpre_out,
                 srcidx_v, dstidx_v, vals_v, rows_v, stage_v, zbuf,
                 acc_sh, cpre_sh, dis_sh, sem, sem2):
    c = lax.axis_index("c")
    s = lax.axis_index("s")
    wid = c * 16 + s
    base = s * ROWS_PER_SUB

    def zrow(i, carry):
        rows_v[i, :] = jnp.zeros((16,), jnp.float32)
        return carry

    lax.fori_loop(0, W, zrow, 0)

    def zsca(i, carry):
        zbuf[pl.ds(i * 16, 16)] = jnp.zeros((16,), jnp.float32)
        return carry

    lax.fori_loop(0, 40, zsca, 0)
    for k in range(4):
        pltpu.sync_copy(rows_v, acc_sh.at[pl.ds(base + k * W, W)])
    pltpu.sync_copy(rows_v.at[pl.ds(0, ROWS_PER_SUB - 4 * W)],
                    acc_sh.at[pl.ds(base + 4 * W, ROWS_PER_SUB - 4 * W)])
    pltpu.sync_copy(zbuf.at[pl.ds(0, ROWS_PER_SUB)],
                    cpre_sh.at[pl.ds(base, ROWS_PER_SUB)])
    pltpu.sync_copy(dis_hbm.at[pl.ds(base, ROWS_PER_SUB)],
                    zbuf.at[pl.ds(0, ROWS_PER_SUB)])
    pltpu.sync_copy(zbuf.at[pl.ds(0, ROWS_PER_SUB)],
                    dis_sh.at[pl.ds(base, ROWS_PER_SUB)])
    pltpu.sync_copy(src_hbm.at[wid], srcidx_v)
    pltpu.sync_copy(dst_hbm.at[wid], dstidx_v)
    plsc.subcore_barrier()

    def body(j, carry):
        a = pltpu.async_copy(hp_hbm.at[srcidx_v.at[j]], rows_v, sem)
        b = pltpu.async_copy(dis_sh.at[dstidx_v.at[j]], vals_v, sem2)
        a.wait()
        b.wait()
        pltpu.sync_copy(rows_v, acc_sh.at[dstidx_v.at[j]], add=True)
        pltpu.sync_copy(vals_v, cpre_sh.at[srcidx_v.at[j]], add=True)
        return carry

    lax.fori_loop(0, K, body, 0)
    plsc.subcore_barrier()
    pltpu.sync_copy(acc_sh.at[pl.ds(base, ROWS_PER_SUB)], stage_v)
    pltpu.sync_copy(stage_v, acc_out.at[pl.ds(c * NPAD + base, ROWS_PER_SUB)])
    pltpu.sync_copy(cpre_sh.at[pl.ds(base, ROWS_PER_SUB)],
                    zbuf.at[pl.ds(0, ROWS_PER_SUB)])
    pltpu.sync_copy(zbuf.at[pl.ds(0, ROWS_PER_SUB)],
                    cpre_out.at[pl.ds(c * NPAD + base, ROWS_PER_SUB)])


def _tc2_body(acc_ref, cpre_ref, hp_ref, dis_ref, b1_ref, w2_ref, b2_ref,
              out_ref):
    dis = dis_ref[...]
    sacc = acc_ref[0] + acc_ref[1] + hp_ref[...]
    out1 = sacc[0:N, :] * dis[0:N, None] + b1_ref[...][None, :]
    r = jnp.maximum(out1, 0.0)
    cpre = cpre_ref[0] + cpre_ref[1] + dis
    cw = dis * cpre
    v = jnp.sum(r * cw[0:N, None], axis=0)
    out = jnp.sum(w2_ref[...] * v[:, None], axis=0) * (1.0 / N) + b2_ref[...]
    out_ref[...] = out


def _tc2(acc_part, cpre_part, hp, dis, b1, w2, b2):
    return pl.pallas_call(
        _tc2_body,
        out_shape=jax.ShapeDtypeStruct((HID,), jnp.float32),
    )(acc_part, cpre_part, hp, dis, b1, w2, b2)


def kernel(neigborhood_state, edges, W1, b1, W2, b2):
    src = edges[0].astype(jnp.int32)
    dst = edges[1].astype(jnp.int32)
    pad = N + (jnp.arange(EPAD - E, dtype=jnp.int32) % (NPAD - N))
    src_p = jnp.concatenate([src, pad]).reshape(NW, K, W)
    dst_p = jnp.concatenate([dst, pad]).reshape(NW, K, W)
    deg_part = _deg_kernel(dst_p).reshape(2, NPAD)
    dis, hp = _tc1(deg_part, neigborhood_state, W1)
    acc_flat, cpre_flat = _main_kernel(src_p, dst_p, hp, dis)
    acc_part = acc_flat.reshape(2, NPAD, HID)
    cpre_part = cpre_flat.reshape(2, NPAD)
    return _tc2(acc_part, cpre_part, hp, dis, b1, W2, b2)


# speedup vs baseline: 99.8673x; 1.6402x over previous
import functools

import jax
import jax.numpy as jnp
from jax import lax
from jax.experimental import pallas as pl
from jax.experimental.pallas import tpu as pltpu
from jax.experimental.pallas import tpu_sc as plsc

N = 10000
E = 320000
HID = 16
NPAD = 10240
NW = 32
W = 128
K = 80
EPAD = NW * K * W
GROUP = 8
NG = K // GROUP
ROWS_PER_SUB = NPAD // 16

_mesh = plsc.VectorSubcoreMesh(core_axis_name="c", subcore_axis_name="s")


@functools.partial(
    pl.kernel,
    out_type=jax.ShapeDtypeStruct((2 * NPAD,), jnp.float32),
    mesh=_mesh,
    scratch_types=[
        pltpu.VMEM((K, W), jnp.int32),
        pltpu.VMEM((W,), jnp.float32),
        pltpu.VMEM((ROWS_PER_SUB,), jnp.float32),
        pltpu.VMEM_SHARED((NPAD,), jnp.float32),
        pltpu.SemaphoreType.DMA,
    ],
    compiler_params=pltpu.CompilerParams(use_tc_tiling_on_sc=False),
)
def _deg_kernel(dst_hbm, deg_out, dstidx_v, ones_v, zbuf, deg_sh, sem):
    c = lax.axis_index("c")
    s = lax.axis_index("s")
    wid = c * 16 + s
    base = s * ROWS_PER_SUB
    for o in range(W // 16):
        ones_v[pl.ds(o * 16, 16)] = jnp.ones((16,), jnp.float32)

    def zstore(i, carry):
        zbuf[pl.ds(i * 16, 16)] = jnp.zeros((16,), jnp.float32)
        return carry

    lax.fori_loop(0, ROWS_PER_SUB // 16, zstore, 0)
    pltpu.sync_copy(zbuf, deg_sh.at[pl.ds(base, ROWS_PER_SUB)])
    pltpu.sync_copy(dst_hbm.at[wid], dstidx_v)
    plsc.subcore_barrier()

    def fire(g):
        for k in range(GROUP):
            pltpu.async_copy(ones_v, deg_sh.at[dstidx_v.at[g * GROUP + k]],
                             sem, add=True)

    def drain(g):
        for k in range(GROUP):
            pltpu.make_async_copy(ones_v,
                                  deg_sh.at[dstidx_v.at[g * GROUP + k]],
                                  sem).wait()

    fire(0)

    def body(g, carry):
        @pl.when(g + 1 < NG)
        def _():
            fire(g + 1)

        drain(g)
        return carry

    lax.fori_loop(0, NG, body, 0)
    plsc.subcore_barrier()
    pltpu.sync_copy(deg_sh.at[pl.ds(base, ROWS_PER_SUB)], zbuf)
    pltpu.sync_copy(zbuf, deg_out.at[pl.ds(c * NPAD + base, ROWS_PER_SUB)])


def _tc1_body(deg_ref, x_ref, w1_ref, dis_ref, hp_ref):
    deg = deg_ref[0, :] + deg_ref[1, :] + 1.0
    dis = lax.rsqrt(deg)
    dis_ref[...] = dis
    h = jnp.dot(x_ref[...], w1_ref[...], preferred_element_type=jnp.float32)
    hp_ref[0:N, :] = h * dis[0:N, None]
    hp_ref[N:NPAD, :] = jnp.zeros((NPAD - N, HID), jnp.float32)


def _tc1(deg_part, x, w1):
    return pl.pallas_call(
        _tc1_body,
        out_shape=(
            jax.ShapeDtypeStruct((NPAD,), jnp.float32),
            jax.ShapeDtypeStruct((NPAD, HID), jnp.float32),
        ),
    )(deg_part, x, w1)


@functools.partial(
    pl.kernel,
    out_type=(
        jax.ShapeDtypeStruct((2 * NPAD, HID), jnp.float32),
        jax.ShapeDtypeStruct((2 * NPAD,), jnp.float32),
    ),
    mesh=_mesh,
    scratch_types=[
        pltpu.VMEM((K, W), jnp.int32),
        pltpu.VMEM((K, W), jnp.int32),
        pltpu.VMEM((2, GROUP * W), jnp.float32),
        pltpu.VMEM((2, GROUP * W, HID), jnp.float32),
        pltpu.VMEM((ROWS_PER_SUB, HID), jnp.float32),
        pltpu.VMEM((ROWS_PER_SUB,), jnp.float32),
        pltpu.VMEM_SHARED((NPAD, HID), jnp.float32),
        pltpu.VMEM_SHARED((NPAD,), jnp.float32),
        pltpu.VMEM_SHARED((NPAD,), jnp.float32),
        pltpu.SemaphoreType.DMA,
        pltpu.SemaphoreType.DMA,
        pltpu.SemaphoreType.DMA,
        pltpu.SemaphoreType.DMA,
    ],
    compiler_params=pltpu.CompilerParams(use_tc_tiling_on_sc=False),
)
def _main_kernel(src_hbm, dst_hbm, hp_hbm, dis_hbm,
                 acc_out, cpre_out,
                 srcidx_v, dstidx_v, vals_v, rows_v, stage_v, zbuf,
                 acc_sh, cpre_sh, dis_sh, sem_gr, sem_gv, sem_sr, sem_sv):
    c = lax.axis_index("c")
    s = lax.axis_index("s")
    wid = c * 16 + s
    base = s * ROWS_PER_SUB

    def zrow(i, carry):
        stage_v[i, :] = jnp.zeros((16,), jnp.float32)
        return carry

    lax.fori_loop(0, ROWS_PER_SUB, zrow, 0)

    def zsca(i, carry):
        zbuf[pl.ds(i * 16, 16)] = jnp.zeros((16,), jnp.float32)
        return carry

    lax.fori_loop(0, ROWS_PER_SUB // 16, zsca, 0)
    pltpu.sync_copy(stage_v, acc_sh.at[pl.ds(base, ROWS_PER_SUB)])
    pltpu.sync_copy(zbuf, cpre_sh.at[pl.ds(base, ROWS_PER_SUB)])
    pltpu.sync_copy(dis_hbm.at[pl.ds(base, ROWS_PER_SUB)], zbuf)
    pltpu.sync_copy(zbuf, dis_sh.at[pl.ds(base, ROWS_PER_SUB)])
    pltpu.sync_copy(src_hbm.at[wid], srcidx_v)
    pltpu.sync_copy(dst_hbm.at[wid], dstidx_v)
    plsc.subcore_barrier()

    def fire_gathers(g, b):
        for k in range(GROUP):
            j = g * GROUP + k
            pltpu.async_copy(hp_hbm.at[srcidx_v.at[j]],
                             rows_v.at[b, pl.ds(k * W, W)], sem_gr)
            pltpu.async_copy(dis_sh.at[dstidx_v.at[j]],
                             vals_v.at[b, pl.ds(k * W, W)], sem_gv)

    def drain_gathers(g, b):
        for k in range(GROUP):
            j = g * GROUP + k
            pltpu.make_async_copy(hp_hbm.at[srcidx_v.at[j]],
                                  rows_v.at[b, pl.ds(k * W, W)], sem_gr).wait()
            pltpu.make_async_copy(dis_sh.at[dstidx_v.at[j]],
                                  vals_v.at[b, pl.ds(k * W, W)], sem_gv).wait()

    def fire_scatters(g, b):
        for k in range(GROUP):
            j = g * GROUP + k
            pltpu.async_copy(rows_v.at[b, pl.ds(k * W, W)],
                             acc_sh.at[dstidx_v.at[j]], sem_sr, add=True)
            pltpu.async_copy(vals_v.at[b, pl.ds(k * W, W)],
                             cpre_sh.at[srcidx_v.at[j]], sem_sv, add=True)

    def drain_scatters(g, b):
        for k in range(GROUP):
            j = g * GROUP + k
            pltpu.make_async_copy(rows_v.at[b, pl.ds(k * W, W)],
                                  acc_sh.at[dstidx_v.at[j]], sem_sr).wait()
            pltpu.make_async_copy(vals_v.at[b, pl.ds(k * W, W)],
                                  cpre_sh.at[srcidx_v.at[j]], sem_sv).wait()

    fire_gathers(0, 0)

    def body(g, carry):
        b = lax.rem(g, 2)
        drain_gathers(g, b)

        @pl.when(g + 1 < NG)
        def _():
            fire_gathers(g + 1, 1 - b)

        fire_scatters(g, b)
        drain_scatters(g, b)
        return carry

    lax.fori_loop(0, NG, body, 0)
    plsc.subcore_barrier()
    pltpu.sync_copy(acc_sh.at[pl.ds(base, ROWS_PER_SUB)], stage_v)
    pltpu.sync_copy(stage_v, acc_out.at[pl.ds(c * NPAD + base, ROWS_PER_SUB)])
    pltpu.sync_copy(cpre_sh.at[pl.ds(base, ROWS_PER_SUB)], zbuf)
    pltpu.sync_copy(zbuf, cpre_out.at[pl.ds(c * NPAD + base, ROWS_PER_SUB)])


def _tc2_body(acc_ref, cpre_ref, hp_ref, dis_ref, b1_ref, w2_ref, b2_ref,
              out_ref):
    dis = dis_ref[...]
    sacc = acc_ref[0] + acc_ref[1] + hp_ref[...]
    out1 = sacc[0:N, :] * dis[0:N, None] + b1_ref[...][None, :]
    r = jnp.maximum(out1, 0.0)
    cpre = cpre_ref[0] + cpre_ref[1] + dis
    cw = dis * cpre
    v = jnp.sum(r * cw[0:N, None], axis=0)
    out = jnp.sum(w2_ref[...] * v[:, None], axis=0) * (1.0 / N) + b2_ref[...]
    out_ref[...] = out


def _tc2(acc_part, cpre_part, hp, dis, b1, w2, b2):
    return pl.pallas_call(
        _tc2_body,
        out_shape=jax.ShapeDtypeStruct((HID,), jnp.float32),
    )(acc_part, cpre_part, hp, dis, b1, w2, b2)


def kernel(neigborhood_state, edges, W1, b1, W2, b2):
    src = edges[0].astype(jnp.int32)
    dst = edges[1].astype(jnp.int32)
    pad = N + (jnp.arange(EPAD - E, dtype=jnp.int32) % (NPAD - N))
    src_p = jnp.concatenate([src, pad]).reshape(NW, K, W)
    dst_p = jnp.concatenate([dst, pad]).reshape(NW, K, W)

    deg_part = _deg_kernel(dst_p).reshape(2, NPAD)
    dis, hp = _tc1(deg_part, neigborhood_state, W1)
    acc_flat, cpre_flat = _main_kernel(src_p, dst_p, hp, dis)
    acc_part = acc_flat.reshape(2, NPAD, HID)
    cpre_part = cpre_flat.reshape(2, NPAD)
    return _tc2(acc_part, cpre_part, hp, dis, b1, W2, b2)
